# Initial kernel scaffold; baseline (speedup 1.0000x reference)
import jax
import jax.numpy as jnp
from jax.experimental import pallas as pl


def kernel(x, edge_index, W1, as1, ad1, b1, W2, as2, ad2, b2, W3, as3, ad3, b3):
    raise NotImplementedError("write your pallas kernel here")


# speedup vs baseline: 20.1058x; 20.1058x over previous
import functools

import jax
import jax.numpy as jnp
from jax import lax
from jax.experimental import pallas as pl
from jax.experimental.pallas import tpu as pltpu
from jax.experimental.pallas import tpu_sc as plsc

N = 10000
E = 160000
D_IN = 256
H1, H2, H3 = 100, 32, 40
HP = 128
NV = HP // 16

BN = 1000
C = 128
NTILES = 32
ROUNDS = -(-(E + N) // (C * NTILES))
EPAD = ROUNDS * C * NTILES
PADN = 112
NACC = N + PADN
ZROWS = NACC // 16
OROWS = 632
OLAST = N - 15 * OROWS


def _tc_first_body(x_ref, w_ref, att_ref, hw_ref, ed_ref, *, h):
    i = pl.program_id(0)
    hh = jnp.dot(x_ref[...], w_ref[...], preferred_element_type=jnp.float32)
    col = lax.broadcasted_iota(jnp.int32, (BN, HP), 1)
    esrc = jnp.sum(hh * att_ref[0:1, :], axis=1, keepdims=True)
    hw_ref[...] = (hh + (col == h).astype(jnp.float32)
                   + esrc * (col == h + 1).astype(jnp.float32))
    ed_ref[i, :] = jnp.sum(hh * att_ref[1:2, :], axis=1)


def _tc_mid_body(acc_ref, b_ref, w_ref, att_ref, hw_ref, ed_ref, *, hprev, h):
    i = pl.program_id(0)
    s = acc_ref[0] + acc_ref[1]
    col = lax.broadcasted_iota(jnp.int32, (BN, HP), 1)
    den = jnp.sum(jnp.where(col == hprev, s, 0.0), axis=1, keepdims=True)
    z = s / (den + 1e-16) + b_ref[0:1, :]
    f = jnp.where(z > 0, z, jnp.exp(jnp.minimum(z, 0.0)) - 1.0)
    f = jnp.where(col < hprev, f, 0.0)
    hh = jnp.dot(f, w_ref[...], preferred_element_type=jnp.float32)
    esrc = jnp.sum(hh * att_ref[0:1, :], axis=1, keepdims=True)
    hw_ref[...] = (hh + (col == h).astype(jnp.float32)
                   + esrc * (col == h + 1).astype(jnp.float32))
    ed_ref[i, :] = jnp.sum(hh * att_ref[1:2, :], axis=1)


def _tc_final_body(acc_ref, b_ref, out_ref, *, hprev):
    s = acc_ref[0] + acc_ref[1]
    col = lax.broadcasted_iota(jnp.int32, (BN, HP), 1)
    den = jnp.sum(jnp.where(col == hprev, s, 0.0), axis=1, keepdims=True)
    z = s / (den + 1e-16) + b_ref[0:1, :]
    zm = jnp.where(col < hprev, z, -3e38)
    m = jnp.max(zm, axis=1, keepdims=True)
    e = jnp.exp(zm - m)
    out_ref[...] = (e / jnp.sum(e, axis=1, keepdims=True))[:, :hprev]


def _call_tc_first(x, w, att, h):
    return pl.pallas_call(
        functools.partial(_tc_first_body, h=h),
        grid=(N // BN,),
        in_specs=[
            pl.BlockSpec((BN, D_IN), lambda i: (i, 0)),
            pl.BlockSpec((D_IN, HP), lambda i: (0, 0)),
            pl.BlockSpec((8, HP), lambda i: (0, 0)),
        ],
        out_specs=[
            pl.BlockSpec((BN, HP), lambda i: (i, 0)),
            pl.BlockSpec((N // BN, BN), lambda i: (0, 0)),
        ],
        out_shape=[
            jax.ShapeDtypeStruct((N, HP), jnp.float32),
            jax.ShapeDtypeStruct((N // BN, BN), jnp.float32),
        ],
    )(x, w, att)


def _call_tc_mid(acc, b, w, att, hprev, h):
    return pl.pallas_call(
        functools.partial(_tc_mid_body, hprev=hprev, h=h),
        grid=(N // BN,),
        in_specs=[
            pl.BlockSpec((2, BN, HP), lambda i: (0, i, 0)),
            pl.BlockSpec((8, HP), lambda i: (0, 0)),
            pl.BlockSpec((HP, HP), lambda i: (0, 0)),
            pl.BlockSpec((8, HP), lambda i: (0, 0)),
        ],
        out_specs=[
            pl.BlockSpec((BN, HP), lambda i: (i, 0)),
            pl.BlockSpec((N // BN, BN), lambda i: (0, 0)),
        ],
        out_shape=[
            jax.ShapeDtypeStruct((N, HP), jnp.float32),
            jax.ShapeDtypeStruct((N // BN, BN), jnp.float32),
        ],
    )(acc, b, w, att)


def _call_tc_final(acc, b, hprev):
    return pl.pallas_call(
        functools.partial(_tc_final_body, hprev=hprev),
        grid=(N // BN,),
        in_specs=[
            pl.BlockSpec((2, BN, HP), lambda i: (0, i, 0)),
            pl.BlockSpec((8, HP), lambda i: (0, 0)),
        ],
        out_specs=pl.BlockSpec((BN, hprev), lambda i: (i, 0)),
        out_shape=jax.ShapeDtypeStruct((N, hprev), jnp.float32),
    )(acc, b)


def _make_edge_kernel(ecol):
    mesh = plsc.VectorSubcoreMesh(core_axis_name="c", subcore_axis_name="s")

    @functools.partial(
        pl.kernel,
        mesh=mesh,
        compiler_params=pltpu.CompilerParams(needs_layout_passes=False),
        out_type=jax.ShapeDtypeStruct((2, N, HP), jnp.float32),
        scratch_types=[
            pltpu.VMEM((1, C), jnp.int32),
            pltpu.VMEM((1, C), jnp.int32),
            pltpu.VMEM((1, C), jnp.int32),
            pltpu.VMEM((C, HP), jnp.float32),
            pltpu.VMEM((C,), jnp.float32),
            pltpu.VMEM((C,), jnp.float32),
            pltpu.VMEM_SHARED((NACC, HP), jnp.float32),
            pltpu.SemaphoreType.DMA,
            pltpu.SemaphoreType.DMA,
        ],
    )
    def edge_kernel(hw_hbm, ed_hbm, src_hbm, dst_hbm, dstg_hbm, out_hbm,
                    isrc_v, idst_v, idstg_v, rows_v, ed_v, w_v, acc,
                    sem, sem2):
        cid = lax.axis_index("c")
        sid = lax.axis_index("s")
        gw = cid * 16 + sid

        zero16 = jnp.zeros((16,), jnp.float32)

        @pl.loop(0, C)
        def _(i):
            for j in range(NV):
                rows_v[i, pl.ds(j * 16, 16)] = zero16

        zbase = sid * ZROWS
        for k in range(4):
            pltpu.sync_copy(rows_v.at[pl.ds(0, C)],
                            acc.at[pl.ds(zbase + k * C, C)])
        pltpu.sync_copy(rows_v.at[pl.ds(0, ZROWS - 4 * C)],
                        acc.at[pl.ds(zbase + 4 * C, ZROWS - 4 * C)])
        plsc.subcore_barrier()

        @pl.loop(0, ROUNDS)
        def _(r):
            pltpu.sync_copy(src_hbm.at[gw, r], isrc_v.at[0])
            pltpu.sync_copy(dst_hbm.at[gw, r], idst_v.at[0])
            pltpu.sync_copy(dstg_hbm.at[gw, r], idstg_v.at[0])
            cp_rows = pltpu.async_copy(hw_hbm.at[isrc_v.at[0]], rows_v, sem)
            cp_ed = pltpu.async_copy(ed_hbm.at[idstg_v.at[0]], ed_v, sem2)
            cp_ed.wait()
            cp_rows.wait()

            for g in range(8):
                riv = jnp.arange(16, dtype=jnp.int32) + (g * 16)
                civ = jnp.full((16,), ecol, jnp.int32)
                es = plsc.load_gather(rows_v, [riv, civ])
                ed = ed_v[pl.ds(g * 16, 16)]
                t = es + ed
                t = jnp.where(t > 0, t, 0.2 * t)
                w_v[pl.ds(g * 16, 16)] = jnp.exp(t)

            @pl.loop(0, C)
            def _(i):
                wv = plsc.load_gather(w_v, [jnp.zeros((16,), jnp.int32) + i])
                for j in range(NV):
                    sl = pl.ds(j * 16, 16)
                    rows_v[i, sl] = rows_v[i, sl] * wv

            pltpu.sync_copy(rows_v, acc.at[idst_v.at[0]], add=True)

        plsc.subcore_barrier()
        ob = sid * OROWS

        @pl.when(sid < 15)
        def _():
            pltpu.sync_copy(acc.at[pl.ds(ob, OROWS)],
                            out_hbm.at[cid, pl.ds(ob, OROWS)])

        @pl.when(sid == 15)
        def _():
            pltpu.sync_copy(acc.at[pl.ds(15 * OROWS, OLAST)],
                            out_hbm.at[cid, pl.ds(15 * OROWS, OLAST)])

    return edge_kernel


_edge_cache = {}


def _edge_k(ecol):
    k = _edge_cache.get(ecol)
    if k is None:
        k = _make_edge_kernel(ecol)
        _edge_cache[ecol] = k
    return k


def _pad2(a, r, c):
    return jnp.pad(a, ((0, r - a.shape[0]), (0, c - a.shape[1])))


def _att8(a_s, a_d):
    z = jnp.zeros((8, HP), jnp.float32)
    z = z.at[0, : a_s.shape[0]].set(a_s)
    return z.at[1, : a_d.shape[0]].set(a_d)


def _b8(b):
    return jnp.zeros((8, HP), jnp.float32).at[0, : b.shape[0]].set(b)


def kernel(x, edge_index, W1, as1, ad1, b1, W2, as2, ad2, b2, W3, as3, ad3, b3):
    ei = edge_index.astype(jnp.int32)
    loop_idx = jnp.arange(N, dtype=jnp.int32)
    npad = EPAD - (E + N)
    pad_src = jnp.arange(npad, dtype=jnp.int32) % 32
    pad_dst = N + (jnp.arange(npad, dtype=jnp.int32) % PADN)
    srcp = jnp.concatenate([ei[0], loop_idx, pad_src]).reshape(NTILES, ROUNDS, C)
    dst_full = jnp.concatenate([ei[1], loop_idx, pad_dst])
    dstp = dst_full.reshape(NTILES, ROUNDS, C)
    dstg = jnp.minimum(dst_full, N - 1).reshape(NTILES, ROUNDS, C)

    w1p = _pad2(W1, D_IN, HP)
    w2p = _pad2(W2, HP, HP)
    w3p = _pad2(W3, HP, HP)

    hw1, ed1 = _call_tc_first(x, w1p, _att8(as1, ad1), H1)
    acc1 = _edge_k(H1 + 1)(hw1, ed1.reshape(N), srcp, dstp, dstg)
    hw2, ed2 = _call_tc_mid(acc1, _b8(b1), w2p, _att8(as2, ad2), H1, H2)
    acc2 = _edge_k(H2 + 1)(hw2, ed2.reshape(N), srcp, dstp, dstg)
    hw3, ed3 = _call_tc_mid(acc2, _b8(b2), w3p, _att8(as3, ad3), H2, H3)
    acc3 = _edge_k(H3 + 1)(hw3, ed3.reshape(N), srcp, dstp, dstg)
    return _call_tc_final(acc3, _b8(b3), H3)


# speedup vs baseline: 35.9085x; 1.7860x over previous
import functools

import jax
import jax.numpy as jnp
from jax import lax
from jax.experimental import pallas as pl
from jax.experimental.pallas import tpu as pltpu
from jax.experimental.pallas import tpu_sc as plsc

N = 10000
E = 160000
D_IN = 256
H1, H2, H3 = 100, 32, 40
HP = 128
NV = HP // 16

BN = 1000
C = 96
NTILES = 32
ROUNDS = -(-(E + N) // (C * NTILES))
EPAD = ROUNDS * C * NTILES
PADN = 112
NACC = N + PADN
ZROWS = NACC // 16
OROWS = 632
OLAST = N - 15 * OROWS


def _tc_first_body(x_ref, w_ref, att_ref, hw_ref, ed_ref, *, h):
    i = pl.program_id(0)
    hh = jnp.dot(x_ref[...], w_ref[...], preferred_element_type=jnp.float32)
    col = lax.broadcasted_iota(jnp.int32, (BN, HP), 1)
    esrc = jnp.sum(hh * att_ref[0:1, :], axis=1, keepdims=True)
    hw_ref[...] = (hh + (col == h).astype(jnp.float32)
                   + esrc * (col == h + 1).astype(jnp.float32))
    ed_ref[i, :] = jnp.sum(hh * att_ref[1:2, :], axis=1)


def _tc_mid_body(acc_ref, b_ref, w_ref, att_ref, hw_ref, ed_ref, *, hprev, h):
    i = pl.program_id(0)
    s = acc_ref[0] + acc_ref[1]
    col = lax.broadcasted_iota(jnp.int32, (BN, HP), 1)
    den = jnp.sum(jnp.where(col == hprev, s, 0.0), axis=1, keepdims=True)
    z = s / (den + 1e-16) + b_ref[0:1, :]
    f = jnp.where(z > 0, z, jnp.exp(jnp.minimum(z, 0.0)) - 1.0)
    f = jnp.where(col < hprev, f, 0.0)
    hh = jnp.dot(f, w_ref[...], preferred_element_type=jnp.float32)
    esrc = jnp.sum(hh * att_ref[0:1, :], axis=1, keepdims=True)
    hw_ref[...] = (hh + (col == h).astype(jnp.float32)
                   + esrc * (col == h + 1).astype(jnp.float32))
    ed_ref[i, :] = jnp.sum(hh * att_ref[1:2, :], axis=1)


def _tc_final_body(acc_ref, b_ref, out_ref, *, hprev):
    s = acc_ref[0] + acc_ref[1]
    col = lax.broadcasted_iota(jnp.int32, (BN, HP), 1)
    den = jnp.sum(jnp.where(col == hprev, s, 0.0), axis=1, keepdims=True)
    z = s / (den + 1e-16) + b_ref[0:1, :]
    zm = jnp.where(col < hprev, z, -3e38)
    m = jnp.max(zm, axis=1, keepdims=True)
    e = jnp.exp(zm - m)
    out_ref[...] = (e / jnp.sum(e, axis=1, keepdims=True))[:, :hprev]


def _call_tc_first(x, w, att, h):
    return pl.pallas_call(
        functools.partial(_tc_first_body, h=h),
        grid=(N // BN,),
        in_specs=[
            pl.BlockSpec((BN, D_IN), lambda i: (i, 0)),
            pl.BlockSpec((D_IN, HP), lambda i: (0, 0)),
            pl.BlockSpec((8, HP), lambda i: (0, 0)),
        ],
        out_specs=[
            pl.BlockSpec((BN, HP), lambda i: (i, 0)),
            pl.BlockSpec((N // BN, BN), lambda i: (0, 0)),
        ],
        out_shape=[
            jax.ShapeDtypeStruct((N, HP), jnp.float32),
            jax.ShapeDtypeStruct((N // BN, BN), jnp.float32),
        ],
    )(x, w, att)


def _call_tc_mid(acc, b, w, att, hprev, h):
    return pl.pallas_call(
        functools.partial(_tc_mid_body, hprev=hprev, h=h),
        grid=(N // BN,),
        in_specs=[
            pl.BlockSpec((2, BN, HP), lambda i: (0, i, 0)),
            pl.BlockSpec((8, HP), lambda i: (0, 0)),
            pl.BlockSpec((HP, HP), lambda i: (0, 0)),
            pl.BlockSpec((8, HP), lambda i: (0, 0)),
        ],
        out_specs=[
            pl.BlockSpec((BN, HP), lambda i: (i, 0)),
            pl.BlockSpec((N // BN, BN), lambda i: (0, 0)),
        ],
        out_shape=[
            jax.ShapeDtypeStruct((N, HP), jnp.float32),
            jax.ShapeDtypeStruct((N // BN, BN), jnp.float32),
        ],
    )(acc, b, w, att)


def _call_tc_final(acc, b, hprev):
    return pl.pallas_call(
        functools.partial(_tc_final_body, hprev=hprev),
        grid=(N // BN,),
        in_specs=[
            pl.BlockSpec((2, BN, HP), lambda i: (0, i, 0)),
            pl.BlockSpec((8, HP), lambda i: (0, 0)),
        ],
        out_specs=pl.BlockSpec((BN, hprev), lambda i: (i, 0)),
        out_shape=jax.ShapeDtypeStruct((N, hprev), jnp.float32),
    )(acc, b)


def _make_edge_kernel(ecol):
    mesh = plsc.VectorSubcoreMesh(core_axis_name="c", subcore_axis_name="s")

    @functools.partial(
        pl.kernel,
        mesh=mesh,
        compiler_params=pltpu.CompilerParams(needs_layout_passes=False),
        out_type=jax.ShapeDtypeStruct((2, N, HP), jnp.float32),
        scratch_types=[
            pltpu.VMEM((ROUNDS, C), jnp.int32),
            pltpu.VMEM((ROUNDS, C), jnp.int32),
            pltpu.VMEM((NACC,), jnp.float32),
            pltpu.VMEM((C, HP), jnp.float32),
            pltpu.VMEM((C, HP), jnp.float32),
            pltpu.VMEM((C,), jnp.float32),
            pltpu.VMEM_SHARED((NACC, HP), jnp.float32),
            pltpu.SemaphoreType.DMA,
            pltpu.SemaphoreType.DMA,
        ],
    )
    def edge_kernel(hw_hbm, ed_hbm, src_hbm, dst_hbm, out_hbm,
                    src_v, dst_v, ed_v, rows0, rows1, w_v, acc,
                    sem0, sem1):
        cid = lax.axis_index("c")
        sid = lax.axis_index("s")
        gw = cid * 16 + sid

        cp_src = pltpu.async_copy(src_hbm.at[gw], src_v, sem0)
        cp_dst = pltpu.async_copy(dst_hbm.at[gw], dst_v, sem1)

        zero16 = jnp.zeros((16,), jnp.float32)

        @pl.loop(0, C)
        def _(i):
            for j in range(NV):
                rows0[i, pl.ds(j * 16, 16)] = zero16

        zbase = sid * ZROWS
        nfull, zrem = ZROWS // C, ZROWS % C
        for k in range(nfull):
            pltpu.sync_copy(rows0.at[pl.ds(0, C)],
                            acc.at[pl.ds(zbase + k * C, C)])
        if zrem:
            pltpu.sync_copy(rows0.at[pl.ds(0, zrem)],
                            acc.at[pl.ds(zbase + nfull * C, zrem)])

        pltpu.sync_copy(ed_hbm, ed_v.at[pl.ds(0, N)])
        for j in range(PADN // 16):
            ed_v[pl.ds(N + j * 16, 16)] = zero16

        cp_src.wait()
        cp_dst.wait()
        plsc.subcore_barrier()

        def compute_scatter(r, buf):
            for g in range(C // 16):
                riv = jnp.arange(16, dtype=jnp.int32) + (g * 16)
                civ = jnp.full((16,), ecol, jnp.int32)
                es = plsc.load_gather(buf, [riv, civ])
                dv = dst_v[r, pl.ds(g * 16, 16)]
                ed = plsc.load_gather(ed_v, [dv])
                t = es + ed
                t = jnp.where(t > 0, t, 0.2 * t)
                w_v[pl.ds(g * 16, 16)] = jnp.exp(t)

            @pl.loop(0, C)
            def _(i):
                wv = plsc.load_gather(w_v, [jnp.zeros((16,), jnp.int32) + i])
                for j in range(NV):
                    sl = pl.ds(j * 16, 16)
                    buf[i, sl] = buf[i, sl] * wv

            pltpu.sync_copy(buf, acc.at[dst_v.at[r]], add=True)

        pltpu.async_copy(hw_hbm.at[src_v.at[0]], rows0, sem0)

        @pl.loop(0, ROUNDS, step=2)
        def _(r):
            cp1 = pltpu.async_copy(hw_hbm.at[src_v.at[r + 1]], rows1, sem1)
            pltpu.make_async_copy(hw_hbm.at[src_v.at[r]], rows0, sem0).wait()
            compute_scatter(r, rows0)

            @pl.when(r + 2 < ROUNDS)
            def _():
                pltpu.async_copy(hw_hbm.at[src_v.at[r + 2]], rows0, sem0)

            cp1.wait()
            compute_scatter(r + 1, rows1)

        plsc.subcore_barrier()
        ob = sid * OROWS

        @pl.when(sid < 15)
        def _():
            pltpu.sync_copy(acc.at[pl.ds(ob, OROWS)],
                            out_hbm.at[cid, pl.ds(ob, OROWS)])

        @pl.when(sid == 15)
        def _():
            pltpu.sync_copy(acc.at[pl.ds(15 * OROWS, OLAST)],
                            out_hbm.at[cid, pl.ds(15 * OROWS, OLAST)])

    return edge_kernel


_edge_cache = {}


def _edge_k(ecol):
    k = _edge_cache.get(ecol)
    if k is None:
        k = _make_edge_kernel(ecol)
        _edge_cache[ecol] = k
    return k


def _pad2(a, r, c):
    return jnp.pad(a, ((0, r - a.shape[0]), (0, c - a.shape[1])))


def _att8(a_s, a_d):
    z = jnp.zeros((8, HP), jnp.float32)
    z = z.at[0, : a_s.shape[0]].set(a_s)
    return z.at[1, : a_d.shape[0]].set(a_d)


def _b8(b):
    return jnp.zeros((8, HP), jnp.float32).at[0, : b.shape[0]].set(b)


def kernel(x, edge_index, W1, as1, ad1, b1, W2, as2, ad2, b2, W3, as3, ad3, b3):
    ei = edge_index.astype(jnp.int32)
    loop_idx = jnp.arange(N, dtype=jnp.int32)
    npad = EPAD - (E + N)
    pad_src = jnp.arange(npad, dtype=jnp.int32) % 32
    pad_dst = N + (jnp.arange(npad, dtype=jnp.int32) % PADN)
    srcp = jnp.concatenate([ei[0], loop_idx, pad_src]).reshape(NTILES, ROUNDS, C)
    dstp = jnp.concatenate([ei[1], loop_idx, pad_dst]).reshape(NTILES, ROUNDS, C)

    w1p = _pad2(W1, D_IN, HP)
    w2p = _pad2(W2, HP, HP)
    w3p = _pad2(W3, HP, HP)

    hw1, ed1 = _call_tc_first(x, w1p, _att8(as1, ad1), H1)
    acc1 = _edge_k(H1 + 1)(hw1, ed1.reshape(N), srcp, dstp)
    hw2, ed2 = _call_tc_mid(acc1, _b8(b1), w2p, _att8(as2, ad2), H1, H2)
    acc2 = _edge_k(H2 + 1)(hw2, ed2.reshape(N), srcp, dstp)
    hw3, ed3 = _call_tc_mid(acc2, _b8(b2), w3p, _att8(as3, ad3), H2, H3)
    acc3 = _edge_k(H3 + 1)(hw3, ed3.reshape(N), srcp, dstp)
    return _call_tc_final(acc3, _b8(b3), H3)


# speedup vs baseline: 38.9673x; 1.0852x over previous
import functools

import jax
import jax.numpy as jnp
from jax import lax
from jax.experimental import pallas as pl
from jax.experimental.pallas import tpu as pltpu
from jax.experimental.pallas import tpu_sc as plsc

N = 10000
E = 160000
D_IN = 256
H1, H2, H3 = 100, 32, 40
HP = 128
NV = HP // 16

BN = 1000
C = 96
NTILES = 32
ROUNDS = -(-(E + N) // (C * NTILES))
EPAD = ROUNDS * C * NTILES
PADN = 112
NACC = N + PADN
ZROWS = NACC // 16
OROWS = 632
OLAST = N - 15 * OROWS


def _tc_first_body(x_ref, w_ref, att_ref, hw_ref, ed_ref, *, h):
    i = pl.program_id(0)
    hh = jnp.dot(x_ref[...], w_ref[...], preferred_element_type=jnp.float32)
    col = lax.broadcasted_iota(jnp.int32, (BN, HP), 1)
    esrc = jnp.sum(hh * att_ref[0:1, :], axis=1, keepdims=True)
    hw_ref[...] = (hh + (col == h).astype(jnp.float32)
                   + esrc * (col == h + 1).astype(jnp.float32))
    ed_ref[i, :] = jnp.sum(hh * att_ref[1:2, :], axis=1)


def _tc_mid_body(acc_ref, b_ref, w_ref, att_ref, hw_ref, ed_ref, *, hprev, h):
    i = pl.program_id(0)
    s = acc_ref[0] + acc_ref[1]
    col = lax.broadcasted_iota(jnp.int32, (BN, HP), 1)
    den = jnp.sum(jnp.where(col == hprev, s, 0.0), axis=1, keepdims=True)
    z = s / (den + 1e-16) + b_ref[0:1, :]
    f = jnp.where(z > 0, z, jnp.exp(jnp.minimum(z, 0.0)) - 1.0)
    f = jnp.where(col < hprev, f, 0.0)
    hh = jnp.dot(f, w_ref[...], preferred_element_type=jnp.float32)
    esrc = jnp.sum(hh * att_ref[0:1, :], axis=1, keepdims=True)
    hw_ref[...] = (hh + (col == h).astype(jnp.float32)
                   + esrc * (col == h + 1).astype(jnp.float32))
    ed_ref[i, :] = jnp.sum(hh * att_ref[1:2, :], axis=1)


def _tc_final_body(acc_ref, b_ref, out_ref, *, hprev):
    s = acc_ref[0] + acc_ref[1]
    col = lax.broadcasted_iota(jnp.int32, (BN, HP), 1)
    den = jnp.sum(jnp.where(col == hprev, s, 0.0), axis=1, keepdims=True)
    z = s / (den + 1e-16) + b_ref[0:1, :]
    zm = jnp.where(col < hprev, z, -3e38)
    m = jnp.max(zm, axis=1, keepdims=True)
    e = jnp.exp(zm - m)
    out_ref[...] = (e / jnp.sum(e, axis=1, keepdims=True))[:, :hprev]


def _call_tc_first(x, w, att, h):
    return pl.pallas_call(
        functools.partial(_tc_first_body, h=h),
        grid=(N // BN,),
        in_specs=[
            pl.BlockSpec((BN, D_IN), lambda i: (i, 0)),
            pl.BlockSpec((D_IN, HP), lambda i: (0, 0)),
            pl.BlockSpec((8, HP), lambda i: (0, 0)),
        ],
        out_specs=[
            pl.BlockSpec((BN, HP), lambda i: (i, 0)),
            pl.BlockSpec((N // BN, BN), lambda i: (0, 0)),
        ],
        out_shape=[
            jax.ShapeDtypeStruct((N, HP), jnp.float32),
            jax.ShapeDtypeStruct((N // BN, BN), jnp.float32),
        ],
    )(x, w, att)


def _call_tc_mid(acc, b, w, att, hprev, h):
    return pl.pallas_call(
        functools.partial(_tc_mid_body, hprev=hprev, h=h),
        grid=(N // BN,),
        in_specs=[
            pl.BlockSpec((2, BN, HP), lambda i: (0, i, 0)),
            pl.BlockSpec((8, HP), lambda i: (0, 0)),
            pl.BlockSpec((HP, HP), lambda i: (0, 0)),
            pl.BlockSpec((8, HP), lambda i: (0, 0)),
        ],
        out_specs=[
            pl.BlockSpec((BN, HP), lambda i: (i, 0)),
            pl.BlockSpec((N // BN, BN), lambda i: (0, 0)),
        ],
        out_shape=[
            jax.ShapeDtypeStruct((N, HP), jnp.float32),
            jax.ShapeDtypeStruct((N // BN, BN), jnp.float32),
        ],
    )(acc, b, w, att)


def _call_tc_final(acc, b, hprev):
    return pl.pallas_call(
        functools.partial(_tc_final_body, hprev=hprev),
        grid=(N // BN,),
        in_specs=[
            pl.BlockSpec((2, BN, HP), lambda i: (0, i, 0)),
            pl.BlockSpec((8, HP), lambda i: (0, 0)),
        ],
        out_specs=pl.BlockSpec((BN, hprev), lambda i: (i, 0)),
        out_shape=jax.ShapeDtypeStruct((N, hprev), jnp.float32),
    )(acc, b)


def _make_edge_kernel(ecol):
    mesh = plsc.VectorSubcoreMesh(core_axis_name="c", subcore_axis_name="s")
    nsl = -(-ecol // 16)

    @functools.partial(
        pl.kernel,
        mesh=mesh,
        compiler_params=pltpu.CompilerParams(needs_layout_passes=False),
        out_type=jax.ShapeDtypeStruct((2, N, HP), jnp.float32),
        scratch_types=[
            pltpu.VMEM((ROUNDS, C), jnp.int32),
            pltpu.VMEM((ROUNDS, C), jnp.int32),
            pltpu.VMEM((NACC,), jnp.float32),
            pltpu.VMEM((C, HP), jnp.float32),
            pltpu.VMEM((C, HP), jnp.float32),
            pltpu.VMEM((C,), jnp.float32),
            pltpu.VMEM_SHARED((NACC, HP), jnp.float32),
            pltpu.SemaphoreType.DMA,
            pltpu.SemaphoreType.DMA,
        ],
    )
    def edge_kernel(hw_hbm, ed_hbm, src_hbm, dst_hbm, out_hbm,
                    src_v, dst_v, ed_v, rows0, rows1, w_v, acc,
                    sem0, sem1):
        cid = lax.axis_index("c")
        sid = lax.axis_index("s")
        gw = cid * 16 + sid

        cp_src = pltpu.async_copy(src_hbm.at[gw], src_v, sem0)
        cp_dst = pltpu.async_copy(dst_hbm.at[gw], dst_v, sem1)

        zero16 = jnp.zeros((16,), jnp.float32)

        @pl.loop(0, C)
        def _(i):
            for j in range(NV):
                rows0[i, pl.ds(j * 16, 16)] = zero16

        zbase = sid * ZROWS
        nfull, zrem = ZROWS // C, ZROWS % C
        for k in range(nfull):
            pltpu.sync_copy(rows0.at[pl.ds(0, C)],
                            acc.at[pl.ds(zbase + k * C, C)])
        if zrem:
            pltpu.sync_copy(rows0.at[pl.ds(0, zrem)],
                            acc.at[pl.ds(zbase + nfull * C, zrem)])

        pltpu.sync_copy(ed_hbm, ed_v.at[pl.ds(0, N)])
        for j in range(PADN // 16):
            ed_v[pl.ds(N + j * 16, 16)] = zero16

        cp_src.wait()
        cp_dst.wait()
        plsc.subcore_barrier()

        def compute_scatter(r, buf):
            for g in range(C // 16):
                riv = jnp.arange(16, dtype=jnp.int32) + (g * 16)
                civ = jnp.full((16,), ecol, jnp.int32)
                es = plsc.load_gather(buf, [riv, civ])
                dv = dst_v[r, pl.ds(g * 16, 16)]
                ed = plsc.load_gather(ed_v, [dv])
                t = es + ed
                t = jnp.where(t > 0, t, 0.2 * t)
                w_v[pl.ds(g * 16, 16)] = jnp.exp(t)

            @pl.loop(0, C)
            def _(i):
                wv = plsc.load_gather(w_v, [jnp.zeros((16,), jnp.int32) + i])
                for j in range(nsl):
                    sl = pl.ds(j * 16, 16)
                    buf[i, sl] = buf[i, sl] * wv

            pltpu.sync_copy(buf, acc.at[dst_v.at[r]], add=True)

        pltpu.async_copy(hw_hbm.at[src_v.at[0]], rows0, sem0)

        @pl.loop(0, ROUNDS, step=2)
        def _(r):
            cp1 = pltpu.async_copy(hw_hbm.at[src_v.at[r + 1]], rows1, sem1)
            pltpu.make_async_copy(hw_hbm.at[src_v.at[r]], rows0, sem0).wait()
            compute_scatter(r, rows0)

            @pl.when(r + 2 < ROUNDS)
            def _():
                pltpu.async_copy(hw_hbm.at[src_v.at[r + 2]], rows0, sem0)

            cp1.wait()
            compute_scatter(r + 1, rows1)

        plsc.subcore_barrier()
        ob = sid * OROWS

        @pl.when(sid < 15)
        def _():
            pltpu.sync_copy(acc.at[pl.ds(ob, OROWS)],
                            out_hbm.at[cid, pl.ds(ob, OROWS)])

        @pl.when(sid == 15)
        def _():
            pltpu.sync_copy(acc.at[pl.ds(15 * OROWS, OLAST)],
                            out_hbm.at[cid, pl.ds(15 * OROWS, OLAST)])

    return edge_kernel


_edge_cache = {}


def _edge_k(ecol):
    k = _edge_cache.get(ecol)
    if k is None:
        k = _make_edge_kernel(ecol)
        _edge_cache[ecol] = k
    return k


def _pad2(a, r, c):
    return jnp.pad(a, ((0, r - a.shape[0]), (0, c - a.shape[1])))


def _att8(a_s, a_d):
    z = jnp.zeros((8, HP), jnp.float32)
    z = z.at[0, : a_s.shape[0]].set(a_s)
    return z.at[1, : a_d.shape[0]].set(a_d)


def _b8(b):
    return jnp.zeros((8, HP), jnp.float32).at[0, : b.shape[0]].set(b)


def kernel(x, edge_index, W1, as1, ad1, b1, W2, as2, ad2, b2, W3, as3, ad3, b3):
    ei = edge_index.astype(jnp.int32)
    loop_idx = jnp.arange(N, dtype=jnp.int32)
    npad = EPAD - (E + N)
    pad_src = jnp.arange(npad, dtype=jnp.int32) % 32
    pad_dst = N + (jnp.arange(npad, dtype=jnp.int32) % PADN)
    srcp = jnp.concatenate([ei[0], loop_idx, pad_src]).reshape(NTILES, ROUNDS, C)
    dstp = jnp.concatenate([ei[1], loop_idx, pad_dst]).reshape(NTILES, ROUNDS, C)

    w1p = _pad2(W1, D_IN, HP)
    w2p = _pad2(W2, HP, HP)
    w3p = _pad2(W3, HP, HP)

    hw1, ed1 = _call_tc_first(x, w1p, _att8(as1, ad1), H1)
    acc1 = _edge_k(H1 + 1)(hw1, ed1.reshape(N), srcp, dstp)
    hw2, ed2 = _call_tc_mid(acc1, _b8(b1), w2p, _att8(as2, ad2), H1, H2)
    acc2 = _edge_k(H2 + 1)(hw2, ed2.reshape(N), srcp, dstp)
    hw3, ed3 = _call_tc_mid(acc2, _b8(b2), w3p, _att8(as3, ad3), H2, H3)
    acc3 = _edge_k(H3 + 1)(hw3, ed3.reshape(N), srcp, dstp)
    return _call_tc_final(acc3, _b8(b3), H3)
